# baseline (speedup 1.0000x reference)
import functools

import jax
import jax.numpy as jnp
from jax import lax
from jax.experimental import pallas as pl
from jax.experimental.pallas import tpu as pltpu
from jax.experimental.pallas import tpu_sc as plsc

B, T, L, V = 1024, 200, 32, 100000
BT = B * T
N_STATIC, N_KNOWN_CAT, N_KNOWN_REAL, N_OBS = 4, 2, 4, 3
KNOWN_F = N_KNOWN_REAL + N_KNOWN_CAT

NC, NS = 2, 16
NW = NC * NS

HU_PER_W = (T * N_KNOWN_CAT * 2) // NW

G_ROWS = T * N_KNOWN_CAT * 256
S_ROWS2 = N_STATIC * 256
KNOWN_ROWS = T * KNOWN_F * 256
OBS_ROWS = T * N_OBS * 256

_mesh = plsc.VectorSubcoreMesh(core_axis_name="c", subcore_axis_name="s")


def _transpose(rows_v, tb):
    lane = lax.iota(jnp.int32, 16)
    i0a = lane // 8
    i1a = lane % 8
    zero = jnp.zeros((16,), jnp.int32)

    def bgl_body(bgl, _2):
        d1b = i1a + bgl * 8

        def blo_body(blo, __):
            r = bgl * 128 + blo
            v0 = rows_v[r, pl.ds(0, 16)]
            v1 = rows_v[r, pl.ds(16, 16)]
            d2 = zero + blo
            plsc.store_scatter(tb, [i0a, d1b, d2], v0)
            plsc.store_scatter(tb, [i0a + 2, d1b, d2], v1)
            return __

        return lax.fori_loop(0, 128, blo_body, _2, unroll=8)

    lax.fori_loop(0, 4, bgl_body, 0)


def _writeback(tb, out_ref, base, sem):
    for lg in range(4):
        pltpu.async_copy(
            tb.at[lg], out_ref.at[pl.ds(base + lg * 64, 32)], sem
        )


def _drain_writes(tb, out_ref, sem):
    for lg in range(4):
        pltpu.make_async_copy(
            tb.at[lg], out_ref.at[pl.ds(lg * 64, 32)], sem
        ).wait()


@functools.partial(
    pl.kernel,
    mesh=_mesh,
    out_type=[jax.ShapeDtypeStruct((G_ROWS, 128), jnp.float32)],
    scratch_types=[
        pltpu.VMEM((3, 8, 2, 128), jnp.int32),
        pltpu.VMEM((3, 512, 32), jnp.float32),
        pltpu.VMEM((2, 4, 32, 128), jnp.float32),
        pltpu.SemaphoreType.DMA,
        pltpu.SemaphoreType.DMA,
    ],
    compiler_params=pltpu.CompilerParams(
        use_tc_tiling_on_sc=False, needs_layout_passes=False),
)
def _sc_gather(ktab, kidx, g_out, idx_t, rows_v, tb, sem_g, sem_o):
    wid = lax.axis_index("s") * NC + lax.axis_index("c")

    def run_cat(tab, w16, ct):
        def stage_a(u, p):
            t = u // 2
            pltpu.sync_copy(kidx.at[t], idx_t.at[p])
            bh = u % 2
            for j in range(4):
                pltpu.async_copy(
                    tab.at[idx_t.at[p, bh * 4 + j, ct]],
                    rows_v.at[p, pl.ds(j * 128, 128)], sem_g,
                )

        def stage_b(u, p, pt, k):
            bh = u % 2
            for j in range(4):
                pltpu.make_async_copy(
                    tab.at[idx_t.at[p, bh * 4 + j, ct]],
                    rows_v.at[p, pl.ds(j * 128, 128)], sem_g,
                ).wait()

            @pl.when(k >= 2)
            def _():
                _drain_writes(tb.at[pt], g_out, sem_o)

            _transpose(rows_v.at[p], tb.at[pt])
            t = u // 2
            _writeback(tb.at[pt], g_out, (t * 2 + ct) * 256 + bh * 32, sem_o)

        u0 = w16 * HU_PER_W
        stage_a(u0, 0)
        stage_a(u0 + 1, 1)

        def unit_body(k, c):
            @pl.when(k + 2 < HU_PER_W)
            def _():
                stage_a(u0 + k + 2, (k + 2) % 3)

            stage_b(u0 + k, k % 3, k % 2, k)
            return c

        lax.fori_loop(0, HU_PER_W, unit_body, 0)
        for pt in range(2):
            _drain_writes(tb.at[pt], g_out, sem_o)

    @pl.when(wid < 16)
    def _():
        run_cat(ktab.at[0], wid, 0)

    @pl.when(wid >= 16)
    def _():
        run_cat(ktab.at[1], wid - 16, 1)


@functools.partial(
    pl.kernel,
    mesh=_mesh,
    out_type=[jax.ShapeDtypeStruct((S_ROWS2, 128), jnp.float32)],
    scratch_types=[
        pltpu.VMEM((512, 32), jnp.float32),
        pltpu.VMEM((4, 32, 128), jnp.float32),
        pltpu.VMEM((8, 4, 128), jnp.int32),
        pltpu.SemaphoreType.DMA,
        pltpu.SemaphoreType.DMA,
    ],
    compiler_params=pltpu.CompilerParams(
        use_tc_tiling_on_sc=False, needs_layout_passes=False),
)
def _sc_static(stab, sidx, s_out, rows_v, tb, sidx_v, sem_g, sem_o):
    wid = lax.axis_index("s") * NC + lax.axis_index("c")
    for f_ in range(N_STATIC):
        @pl.when(wid // 2 == f_)
        def _(f_=f_):
            bh = wid % 2
            pltpu.sync_copy(sidx, sidx_v)
            tab = stab.at[f_]
            for j in range(4):
                pltpu.async_copy(
                    tab.at[sidx_v.at[bh * 4 + j, f_]],
                    rows_v.at[pl.ds(j * 128, 128)], sem_g,
                )
            for j in range(4):
                pltpu.make_async_copy(
                    tab.at[sidx_v.at[bh * 4 + j, f_]],
                    rows_v.at[pl.ds(j * 128, 128)], sem_g,
                ).wait()
            _transpose(rows_v, tb)
            _writeback(tb, s_out, f_ * 256 + bh * 32, sem_o)
            _drain_writes(tb, s_out, sem_o)


def _known_body(kr_ref, g_ref, w_ref, b_ref, o_ref):
    kr = kr_ref[...]
    krt = kr.reshape(8, 4, 128).transpose(1, 0, 2)
    kr_exp = jnp.broadcast_to(
        krt.reshape(4, 1, 8, 1, 128), (4, 4, 8, 8, 128)
    ).reshape(1024, 128)
    o_ref[pl.ds(0, 1024), :] = w_ref[...] * kr_exp + b_ref[...]
    o_ref[pl.ds(1024, 512), :] = g_ref[...]


def _obs_body(x_ref, w_ref, b_ref, o_ref):
    x = x_ref[...]
    xt = x.reshape(3, 8, 8, 128).transpose(2, 0, 1, 3)
    x_exp = jnp.broadcast_to(
        xt.reshape(8, 3, 1, 8, 1, 128), (8, 3, 4, 8, 8, 128)
    ).reshape(6144, 128)
    o_ref[...] = w_ref[...] * x_exp + b_ref[...]


def kernel(static, known_real, known_categorical, observed, static_tables,
           known_tables, known_dense_w, known_dense_b, observed_dense_w,
           observed_dense_b):
    f32, i32 = jnp.float32, jnp.int32

    kidxN = (known_categorical.astype(i32)
             .reshape(8, 128, T, N_KNOWN_CAT).transpose(2, 0, 3, 1))
    sidxN = (static[:, 0, :].astype(i32)
             .reshape(8, 128, N_STATIC).transpose(0, 2, 1))
    krN = (known_real.reshape(8, 128, T, N_KNOWN_REAL)
           .transpose(2, 0, 3, 1).reshape(T * 32, 128))
    obsN = (observed.reshape(8, 128, 25, 8, N_OBS)
            .transpose(4, 2, 0, 3, 1))

    ktab = known_tables
    stab = static_tables

    w = known_dense_w.reshape(N_KNOWN_REAL, L)
    bw = known_dense_b.reshape(N_KNOWN_REAL, L)
    w_big = jnp.broadcast_to(
        w.reshape(4, 4, 1, 8, 1), (4, 4, 8, 8, 128)).reshape(1024, 128)
    b_big = jnp.broadcast_to(
        bw.reshape(4, 4, 1, 8, 1), (4, 4, 8, 8, 128)).reshape(1024, 128)
    wo = observed_dense_w.reshape(N_OBS, L)
    bo = observed_dense_b.reshape(N_OBS, L)
    wo_big = jnp.broadcast_to(
        wo.reshape(1, 3, 4, 1, 8, 1), (8, 3, 4, 8, 8, 128)).reshape(6144, 128)
    bo_big = jnp.broadcast_to(
        bo.reshape(1, 3, 4, 1, 8, 1), (8, 3, 4, 8, 8, 128)).reshape(6144, 128)

    (g2,) = _sc_gather(ktab, kidxN)
    (s2,) = _sc_static(stab, sidxN)

    out_o = pl.pallas_call(
        _obs_body,
        grid=(25,),
        in_specs=[
            pl.BlockSpec((3, 1, 8, 8, 128), lambda i: (0, i, 0, 0, 0)),
            pl.BlockSpec((6144, 128), lambda i: (0, 0)),
            pl.BlockSpec((6144, 128), lambda i: (0, 0)),
        ],
        out_specs=pl.BlockSpec((6144, 128), lambda i: (i, 0)),
        out_shape=jax.ShapeDtypeStruct((OBS_ROWS, 128), f32),
    )(obsN, wo_big, bo_big)

    out2 = pl.pallas_call(
        _known_body,
        grid=(T,),
        in_specs=[
            pl.BlockSpec((32, 128), lambda i: (i, 0)),
            pl.BlockSpec((512, 128), lambda i: (i, 0)),
            pl.BlockSpec((1024, 128), lambda i: (0, 0)),
            pl.BlockSpec((1024, 128), lambda i: (0, 0)),
        ],
        out_specs=pl.BlockSpec((1536, 128), lambda i: (i, 0)),
        out_shape=jax.ShapeDtypeStruct((KNOWN_ROWS, 128), f32),
    )(krN, g2, w_big, b_big)

    known = (out2.reshape(T, KNOWN_F, 4, 8, 8, 128)
             .transpose(3, 5, 0, 2, 4, 1).reshape(B, T, L, KNOWN_F))
    observed_embeds = (out_o.reshape(T, N_OBS, 4, 8, 8, 128)
                       .transpose(3, 5, 0, 2, 4, 1).reshape(B, T, L, N_OBS))
    static_embeds = (s2.reshape(N_STATIC, 4, 8, 8, 128)
                     .transpose(2, 4, 0, 1, 3).reshape(B, N_STATIC, L))
    return (static_embeds, known, observed_embeds)


# speedup vs baseline: 1.1755x; 1.1755x over previous
import functools

import jax
import jax.numpy as jnp
from jax import lax
from jax.experimental import pallas as pl
from jax.experimental.pallas import tpu as pltpu
from jax.experimental.pallas import tpu_sc as plsc

B, T, L, V = 1024, 200, 32, 100000
BT = B * T
N_STATIC, N_KNOWN_CAT, N_KNOWN_REAL, N_OBS = 4, 2, 4, 3
KNOWN_F = N_KNOWN_REAL + N_KNOWN_CAT

NC, NS = 2, 16
NW = NC * NS

HU_PER_W = (T * N_KNOWN_CAT * 2) // NW

G_ROWS = T * N_KNOWN_CAT * 256
S_ROWS2 = N_STATIC * 256
KNOWN_ROWS = T * KNOWN_F * 256
OBS_ROWS = T * N_OBS * 256

_mesh = plsc.VectorSubcoreMesh(core_axis_name="c", subcore_axis_name="s")


def _transpose(rows_v, tb):
    lane = lax.iota(jnp.int32, 16)
    i0a = lane // 8
    i1a = lane % 8
    zero = jnp.zeros((16,), jnp.int32)

    def bgl_body(bgl, _2):
        d1b = i1a + bgl * 8

        def blo_body(blo, __):
            r = bgl * 128 + blo
            v0 = rows_v[r, pl.ds(0, 16)]
            v1 = rows_v[r, pl.ds(16, 16)]
            d2 = zero + blo
            plsc.store_scatter(tb, [i0a, d1b, d2], v0)
            plsc.store_scatter(tb, [i0a + 2, d1b, d2], v1)
            return __

        return lax.fori_loop(0, 128, blo_body, _2, unroll=8)

    lax.fori_loop(0, 4, bgl_body, 0)


def _writeback(tb, out_ref, base, sem):
    for lg in range(4):
        pltpu.async_copy(
            tb.at[lg, pl.ds(0, 32), pl.ds(0, 128)],
            out_ref.at[pl.ds(base + lg * 64, 32)], sem,
        )


def _drain_writes(tb, out_ref, sem):
    for lg in range(4):
        pltpu.make_async_copy(
            tb.at[lg, pl.ds(0, 32), pl.ds(0, 128)],
            out_ref.at[pl.ds(lg * 64, 32)], sem,
        ).wait()


@functools.partial(
    pl.kernel,
    mesh=_mesh,
    out_type=[jax.ShapeDtypeStruct((G_ROWS, 128), jnp.float32)],
    scratch_types=[
        pltpu.VMEM((3, 8, 2, 128), jnp.int32),
        pltpu.VMEM((3, 512, 32), jnp.float32),
        pltpu.VMEM((2, 4, 33, 129), jnp.float32),
        pltpu.SemaphoreType.DMA,
        pltpu.SemaphoreType.DMA,
    ],
    compiler_params=pltpu.CompilerParams(
        use_tc_tiling_on_sc=False, needs_layout_passes=False),
)
def _sc_gather(ktab, kidx, g_out, idx_t, rows_v, tb, sem_g, sem_o):
    wid = lax.axis_index("s") * NC + lax.axis_index("c")

    def run_cat(tab, w16, ct):
        def stage_a(u, p):
            t = u // 2
            pltpu.sync_copy(kidx.at[t], idx_t.at[p])
            bh = u % 2
            for j in range(4):
                pltpu.async_copy(
                    tab.at[idx_t.at[p, bh * 4 + j, ct]],
                    rows_v.at[p, pl.ds(j * 128, 128)], sem_g,
                )

        def stage_b(u, p, pt, k):
            bh = u % 2
            for j in range(4):
                pltpu.make_async_copy(
                    tab.at[idx_t.at[p, bh * 4 + j, ct]],
                    rows_v.at[p, pl.ds(j * 128, 128)], sem_g,
                ).wait()

            @pl.when(k >= 2)
            def _():
                _drain_writes(tb.at[pt], g_out, sem_o)

            _transpose(rows_v.at[p], tb.at[pt])
            t = u // 2
            _writeback(tb.at[pt], g_out, (t * 2 + ct) * 256 + bh * 32, sem_o)

        u0 = w16 * HU_PER_W
        stage_a(u0, 0)
        stage_a(u0 + 1, 1)

        def unit_body(k, c):
            @pl.when(k + 2 < HU_PER_W)
            def _():
                stage_a(u0 + k + 2, (k + 2) % 3)

            stage_b(u0 + k, k % 3, k % 2, k)
            return c

        lax.fori_loop(0, HU_PER_W, unit_body, 0)
        for pt in range(2):
            _drain_writes(tb.at[pt], g_out, sem_o)

    @pl.when(wid < 16)
    def _():
        run_cat(ktab.at[0], wid, 0)

    @pl.when(wid >= 16)
    def _():
        run_cat(ktab.at[1], wid - 16, 1)


@functools.partial(
    pl.kernel,
    mesh=_mesh,
    out_type=[jax.ShapeDtypeStruct((S_ROWS2, 128), jnp.float32)],
    scratch_types=[
        pltpu.VMEM((512, 32), jnp.float32),
        pltpu.VMEM((4, 33, 129), jnp.float32),
        pltpu.VMEM((8, 4, 128), jnp.int32),
        pltpu.SemaphoreType.DMA,
        pltpu.SemaphoreType.DMA,
    ],
    compiler_params=pltpu.CompilerParams(
        use_tc_tiling_on_sc=False, needs_layout_passes=False),
)
def _sc_static(stab, sidx, s_out, rows_v, tb, sidx_v, sem_g, sem_o):
    wid = lax.axis_index("s") * NC + lax.axis_index("c")
    for f_ in range(N_STATIC):
        @pl.when(wid // 2 == f_)
        def _(f_=f_):
            bh = wid % 2
            pltpu.sync_copy(sidx, sidx_v)
            tab = stab.at[f_]
            for j in range(4):
                pltpu.async_copy(
                    tab.at[sidx_v.at[bh * 4 + j, f_]],
                    rows_v.at[pl.ds(j * 128, 128)], sem_g,
                )
            for j in range(4):
                pltpu.make_async_copy(
                    tab.at[sidx_v.at[bh * 4 + j, f_]],
                    rows_v.at[pl.ds(j * 128, 128)], sem_g,
                ).wait()
            _transpose(rows_v, tb)
            _writeback(tb, s_out, f_ * 256 + bh * 32, sem_o)
            _drain_writes(tb, s_out, sem_o)


def _known_body(kr_ref, g_ref, w_ref, b_ref, o_ref):
    kr = kr_ref[...]
    krt = kr.reshape(8, 4, 128).transpose(1, 0, 2)
    kr_exp = jnp.broadcast_to(
        krt.reshape(4, 1, 8, 1, 128), (4, 4, 8, 8, 128)
    ).reshape(1024, 128)
    o_ref[pl.ds(0, 1024), :] = w_ref[...] * kr_exp + b_ref[...]
    o_ref[pl.ds(1024, 512), :] = g_ref[...]


def _obs_body(x_ref, w_ref, b_ref, o_ref):
    x = x_ref[...]
    xt = x.reshape(3, 8, 8, 128).transpose(2, 0, 1, 3)
    x_exp = jnp.broadcast_to(
        xt.reshape(8, 3, 1, 8, 1, 128), (8, 3, 4, 8, 8, 128)
    ).reshape(6144, 128)
    o_ref[...] = w_ref[...] * x_exp + b_ref[...]


def kernel(static, known_real, known_categorical, observed, static_tables,
           known_tables, known_dense_w, known_dense_b, observed_dense_w,
           observed_dense_b):
    f32, i32 = jnp.float32, jnp.int32

    kidxN = (known_categorical.astype(i32)
             .reshape(8, 128, T, N_KNOWN_CAT).transpose(2, 0, 3, 1))
    sidxN = (static[:, 0, :].astype(i32)
             .reshape(8, 128, N_STATIC).transpose(0, 2, 1))
    krN = (known_real.reshape(8, 128, T, N_KNOWN_REAL)
           .transpose(2, 0, 3, 1).reshape(T * 32, 128))
    obsN = (observed.reshape(8, 128, 25, 8, N_OBS)
            .transpose(4, 2, 0, 3, 1))

    ktab = known_tables
    stab = static_tables

    w = known_dense_w.reshape(N_KNOWN_REAL, L)
    bw = known_dense_b.reshape(N_KNOWN_REAL, L)
    w_big = jnp.broadcast_to(
        w.reshape(4, 4, 1, 8, 1), (4, 4, 8, 8, 128)).reshape(1024, 128)
    b_big = jnp.broadcast_to(
        bw.reshape(4, 4, 1, 8, 1), (4, 4, 8, 8, 128)).reshape(1024, 128)
    wo = observed_dense_w.reshape(N_OBS, L)
    bo = observed_dense_b.reshape(N_OBS, L)
    wo_big = jnp.broadcast_to(
        wo.reshape(1, 3, 4, 1, 8, 1), (8, 3, 4, 8, 8, 128)).reshape(6144, 128)
    bo_big = jnp.broadcast_to(
        bo.reshape(1, 3, 4, 1, 8, 1), (8, 3, 4, 8, 8, 128)).reshape(6144, 128)

    (g2,) = _sc_gather(ktab, kidxN)
    (s2,) = _sc_static(stab, sidxN)

    out_o = pl.pallas_call(
        _obs_body,
        grid=(25,),
        in_specs=[
            pl.BlockSpec((3, 1, 8, 8, 128), lambda i: (0, i, 0, 0, 0)),
            pl.BlockSpec((6144, 128), lambda i: (0, 0)),
            pl.BlockSpec((6144, 128), lambda i: (0, 0)),
        ],
        out_specs=pl.BlockSpec((6144, 128), lambda i: (i, 0)),
        out_shape=jax.ShapeDtypeStruct((OBS_ROWS, 128), f32),
    )(obsN, wo_big, bo_big)

    out2 = pl.pallas_call(
        _known_body,
        grid=(T,),
        in_specs=[
            pl.BlockSpec((32, 128), lambda i: (i, 0)),
            pl.BlockSpec((512, 128), lambda i: (i, 0)),
            pl.BlockSpec((1024, 128), lambda i: (0, 0)),
            pl.BlockSpec((1024, 128), lambda i: (0, 0)),
        ],
        out_specs=pl.BlockSpec((1536, 128), lambda i: (i, 0)),
        out_shape=jax.ShapeDtypeStruct((KNOWN_ROWS, 128), f32),
    )(krN, g2, w_big, b_big)

    known = (out2.reshape(T, KNOWN_F, 4, 8, 8, 128)
             .transpose(3, 5, 0, 2, 4, 1).reshape(B, T, L, KNOWN_F))
    observed_embeds = (out_o.reshape(T, N_OBS, 4, 8, 8, 128)
                       .transpose(3, 5, 0, 2, 4, 1).reshape(B, T, L, N_OBS))
    static_embeds = (s2.reshape(N_STATIC, 4, 8, 8, 128)
                     .transpose(2, 4, 0, 1, 3).reshape(B, N_STATIC, L))
    return (static_embeds, known, observed_embeds)
